# baseline (speedup 1.0000x reference)
import jax
import jax.numpy as jnp
from jax import lax
from jax.experimental import pallas as pl
from jax.experimental.pallas import tpu as pltpu
from jax.experimental.pallas import tpu_sc as plsc

_N = 10000
_E = 160000
_NC = 2
_NS = 16
_K = 128
_EPT = 10240
_EPAD = _EPT * _NS
_NBATCH = _EPT // _K
_NP = 10240
_ROWS = _NP // _NS


def _make_sc_agg(n_chunks):
  cps = n_chunks // _NC
  mesh = plsc.VectorSubcoreMesh(core_axis_name="c", subcore_axis_name="s")

  slab = 2 * _K
  nslabs = _EPT // slab
  groups = ((0, 24), (24, 16))

  scratch = [
      pltpu.VMEM((24 * slab,), jnp.int32),
      pltpu.VMEM((48, _K), jnp.int32),
      pltpu.VMEM((slab, 128), jnp.float32),
      pltpu.VMEM_SHARED((_NP, 128), jnp.float32),
      pltpu.SemaphoreType.DMA,
      pltpu.SemaphoreType.DMA,
      pltpu.SemaphoreType.DMA,
  ]

  def body(x2, src_off, dst2, zeros, out2, src_all, dst_all,
           rows, acc, gsem, ssem0, ssem1):
    cid = lax.axis_index("c")
    sid = lax.axis_index("s")
    r0 = sid * _ROWS
    e0 = sid * _EPT

    for ci in range(cps):
      c = cid * cps + ci

      pltpu.sync_copy(zeros.at[pl.ds(r0, _ROWS)], acc.at[pl.ds(r0, _ROWS)])
      plsc.subcore_barrier()

      for off, n in groups:
        r_base = (c * (_EPAD // slab) + sid * nslabs + off) * slab
        pltpu.sync_copy(src_off.at[pl.ds(r_base, n * slab)],
                        src_all.at[pl.ds(0, n * slab)])
        pltpu.sync_copy(dst2.at[pl.ds(sid * _NBATCH + 2 * off, 2 * n)],
                        dst_all.at[pl.ds(0, 2 * n)])
        pltpu.async_copy(x2.at[src_all.at[pl.ds(0, slab)]], rows, gsem)

        def step(g, carry):
          pltpu.make_async_copy(x2.at[src_all.at[pl.ds(0, slab)]], rows,
                                gsem).wait()
          pltpu.async_copy(rows.at[pl.ds(0, _K)],
                           acc.at[dst_all.at[2 * g]], ssem0, add=True)
          pltpu.async_copy(rows.at[pl.ds(_K, _K)],
                           acc.at[dst_all.at[2 * g + 1]], ssem1, add=True)
          pltpu.make_async_copy(rows.at[pl.ds(0, _K)],
                                acc.at[dst_all.at[0]], ssem0).wait()
          pltpu.make_async_copy(rows.at[pl.ds(0, _K)],
                                acc.at[dst_all.at[0]], ssem1).wait()
          nxt = jnp.minimum(g + 1, n - 1) * slab
          pltpu.async_copy(x2.at[src_all.at[pl.ds(nxt, slab)]], rows, gsem)
          return carry

        lax.fori_loop(0, n, step, 0)
        pltpu.make_async_copy(x2.at[src_all.at[pl.ds(0, slab)]], rows,
                              gsem).wait()
      plsc.subcore_barrier()

      pltpu.sync_copy(acc.at[pl.ds(r0, _ROWS)],
                      out2.at[pl.ds(c * _NP + r0, _ROWS)])

  return pl.kernel(
      body,
      out_type=jax.ShapeDtypeStruct((n_chunks * _NP, 128), jnp.float32),
      mesh=mesh, scratch_types=scratch)


def _make_sc_counts():
  mesh = plsc.VectorSubcoreMesh(core_axis_name="c", subcore_axis_name="s")

  scratch = [
      pltpu.VMEM((_NBATCH, _K), jnp.int32),
      pltpu.VMEM((_K, 128), jnp.float32),
      pltpu.VMEM_SHARED((_NP, 128), jnp.float32),
      pltpu.SemaphoreType.DMA,
      pltpu.SemaphoreType.DMA,
  ]

  def body(dst2, zeros, ones, out2, dst_all, ones_v, acc, sem0, sem1):
    sid = lax.axis_index("s")
    r0 = sid * _ROWS
    sems = (sem0, sem1)

    pltpu.sync_copy(ones, ones_v)
    pltpu.sync_copy(dst2.at[pl.ds(sid * _NBATCH, _NBATCH)], dst_all)
    pltpu.sync_copy(zeros.at[pl.ds(r0, _ROWS)], acc.at[pl.ds(r0, _ROWS)])
    plsc.subcore_barrier()

    for j in range(2):
      pltpu.async_copy(ones_v, acc.at[dst_all.at[j]], sems[j], add=True)

    def step(g, carry):
      for j in range(2):
        b = 2 * g + j
        pltpu.make_async_copy(ones_v, acc.at[dst_all.at[0]],
                              sems[j]).wait()
        pltpu.async_copy(ones_v, acc.at[dst_all.at[b + 2]],
                         sems[j], add=True)
      return carry

    lax.fori_loop(0, _NBATCH // 2 - 1, step, 0)
    for j in range(2):
      pltpu.make_async_copy(ones_v, acc.at[dst_all.at[0]], sems[j]).wait()
    plsc.subcore_barrier()
    pltpu.sync_copy(acc.at[pl.ds(r0, _ROWS)], out2.at[pl.ds(r0, _ROWS)])

  return pl.kernel(
      body,
      out_type=jax.ShapeDtypeStruct((_NP, 128), jnp.float32),
      mesh=mesh, scratch_types=scratch)


def _make_tc_layer(c_in, residual, n_blk=512):
  d_in = c_in * 128
  grid = (_NP // n_blk,)

  def body(agg_r, x_r, cnt_r, wl_r, wr_r, b_r, out_r):
    inv = 1.0 / jnp.maximum(cnt_r[:, 0:1], 1.0)
    agg = jnp.concatenate([agg_r[i] for i in range(c_in)], axis=1)
    xin = jnp.concatenate([x_r[i] for i in range(c_in)], axis=1)
    z = jnp.dot(agg * inv, wl_r[...], preferred_element_type=jnp.float32)
    z = z + jnp.dot(xin, wr_r[...], preferred_element_type=jnp.float32)
    z = jnp.maximum(z + b_r[...], 0.0)
    if residual:
      z = z + xin
    for i in range(4):
      out_r[i] = z[:, 128 * i:128 * (i + 1)]

  in_specs = [
      pl.BlockSpec((c_in, n_blk, 128), lambda i: (0, i, 0)),
      pl.BlockSpec((c_in, n_blk, 128), lambda i: (0, i, 0)),
      pl.BlockSpec((n_blk, 128), lambda i: (i, 0)),
      pl.BlockSpec((d_in, 512), lambda i: (0, 0)),
      pl.BlockSpec((d_in, 512), lambda i: (0, 0)),
      pl.BlockSpec((1, 512), lambda i: (0, 0)),
  ]
  return pl.pallas_call(
      body,
      grid=grid,
      in_specs=in_specs,
      out_specs=pl.BlockSpec((4, n_blk, 128), lambda i: (0, i, 0)),
      out_shape=jax.ShapeDtypeStruct((4, _NP, 128), jnp.float32),
  )


def _make_tc_head():
  def body(h_r, w1_r, b1_r, w2_r, b2_r, w3_r, b3_r, out_r):
    s = jnp.sum(h_r[:, 0:_N, :], axis=1) * (1.0 / _N)
    z1 = jnp.zeros((1, 256), jnp.float32)
    for c in range(4):
      z1 = z1 + jnp.dot(s[c:c + 1], w1_r[c], preferred_element_type=jnp.float32)
    z1 = jnp.maximum(z1 + b1_r[...], 0.0)
    z2 = jnp.dot(z1, w2_r[...], preferred_element_type=jnp.float32)
    z2 = jnp.maximum(z2 + b2_r[...], 0.0)
    z3 = jnp.dot(z2, w3_r[...], preferred_element_type=jnp.float32) + b3_r[...]
    out_r[...] = 1.0 / (1.0 + jnp.exp(-z3))

  return pl.pallas_call(
      body, out_shape=jax.ShapeDtypeStruct((1, 1), jnp.float32))


_sc_agg2 = _make_sc_agg(2)
_sc_agg4 = _make_sc_agg(4)
_sc_counts = _make_sc_counts()
_tc_layer0 = _make_tc_layer(2, residual=False)
_tc_layer_res = _make_tc_layer(4, residual=True)
_tc_head = _make_tc_head()


def _chunk_offsets(src_p, n_chunks):
  return (src_p[None, :] +
          (jnp.arange(n_chunks, dtype=jnp.int32) * _NP)[:, None]).reshape(-1)


@jax.jit
def kernel(x, edge_index, Wl0, bl0, Wr0, Wl1, bl1, Wr1, Wl2, bl2, Wr2,
           Wc1, bc1, Wc2, bc2, Wc3, bc3):
  src = edge_index[0].astype(jnp.int32)
  dst = edge_index[1].astype(jnp.int32)
  pad = _EPAD - _E
  src_p = jnp.concatenate([src, jnp.zeros((pad,), jnp.int32)])
  dst_p = jnp.concatenate([dst, jnp.full((pad,), _N, jnp.int32)])
  src_off2 = _chunk_offsets(src_p, 2)
  src_off4 = _chunk_offsets(src_p, 4)

  x3 = x.reshape(_N, 2, 128).transpose(1, 0, 2)
  x3 = jnp.pad(x3, ((0, 0), (0, _NP - _N), (0, 0)))
  zeros = jnp.zeros((_NP, 128), jnp.float32)
  ones = jnp.ones((_K, 128), jnp.float32)

  dst2a = dst_p.reshape(_EPAD // _K, _K)
  cnt = _sc_counts(dst2a, zeros, ones)
  agg0 = _sc_agg2(x3.reshape(2 * _NP, 128), src_off2, dst2a, zeros)
  h = _tc_layer0(agg0.reshape(2, _NP, 128), x3, cnt, Wl0.T, Wr0.T,
                 bl0[None, :])

  agg1 = _sc_agg4(h.reshape(4 * _NP, 128), src_off4, dst2a, zeros)
  h = _tc_layer_res(agg1.reshape(4, _NP, 128), h, cnt, Wl1.T, Wr1.T,
                    bl1[None, :])

  agg2 = _sc_agg4(h.reshape(4 * _NP, 128), src_off4, dst2a, zeros)
  h = _tc_layer_res(agg2.reshape(4, _NP, 128), h, cnt, Wl2.T, Wr2.T,
                    bl2[None, :])

  return _tc_head(h, Wc1.T.reshape(4, 128, 256), bc1[None, :],
                  Wc2.T, bc2[None, :], Wc3.T, bc3[None, :])


# speedup vs baseline: 1.1710x; 1.1710x over previous
import jax
import jax.numpy as jnp
from jax import lax
from jax.experimental import pallas as pl
from jax.experimental.pallas import tpu as pltpu
from jax.experimental.pallas import tpu_sc as plsc

_N = 10000
_E = 160000
_NC = 2
_NS = 16
_K = 128
_EPT = 10240
_EPAD = _EPT * _NS
_NBATCH = _EPT // _K
_NP = 10240
_ROWS = _NP // _NS


def _make_sc_agg(n_chunks):
  cps = n_chunks // _NC
  mesh = plsc.VectorSubcoreMesh(core_axis_name="c", subcore_axis_name="s")

  scratch = [
      pltpu.VMEM((_EPT // 2 + 2 * _K,), jnp.int32),
      pltpu.VMEM((_NBATCH, _K), jnp.int32),
      pltpu.VMEM((_K, 128), jnp.float32),
      pltpu.VMEM((_K, 128), jnp.float32),
      pltpu.VMEM_SHARED((_NP, 128), jnp.float32),
      pltpu.SemaphoreType.DMA,
      pltpu.SemaphoreType.DMA,
  ]

  def body(x2, src_off, dst2, zeros, out2, src_all, dst_all,
           rows0, rows1, acc, gsem0, gsem1):
    cid = lax.axis_index("c")
    sid = lax.axis_index("s")
    r0 = sid * _ROWS
    e0 = sid * _EPT
    bufs = ((rows0, gsem0), (rows1, gsem1))

    pltpu.sync_copy(dst2.at[pl.ds(sid * _NBATCH, _NBATCH)], dst_all)

    hept = _EPT // 2
    hnb = _NBATCH // 2

    for ci in range(cps):
      c = cid * cps + ci

      pltpu.sync_copy(zeros.at[pl.ds(r0, _ROWS)], acc.at[pl.ds(r0, _ROWS)])
      plsc.subcore_barrier()

      for h in range(2):
        pltpu.sync_copy(
            src_off.at[pl.ds(c * _EPAD + e0 + h * hept, hept + 2 * _K)],
            src_all)

        for j in range(2):
          rows, gsem = bufs[j]
          pltpu.async_copy(x2.at[src_all.at[pl.ds(j * _K, _K)]], rows, gsem)

        def step(g, carry):
          for j in range(2):
            bl = 2 * g + j
            rows, gsem = bufs[j]
            pltpu.make_async_copy(x2.at[src_all.at[pl.ds(0, _K)]],
                                  rows, gsem).wait()
            pltpu.sync_copy(rows, acc.at[dst_all.at[h * hnb + bl]],
                            add=True)
            pltpu.async_copy(x2.at[src_all.at[pl.ds((bl + 2) * _K, _K)]],
                             rows, gsem)
          return carry

        lax.fori_loop(0, hnb // 2, step, 0)
        for j in range(2):
          rows, gsem = bufs[j]
          pltpu.make_async_copy(x2.at[src_all.at[pl.ds(0, _K)]],
                                rows, gsem).wait()
      plsc.subcore_barrier()

      pltpu.sync_copy(acc.at[pl.ds(r0, _ROWS)],
                      out2.at[pl.ds(c * _NP + r0, _ROWS)])

  return pl.kernel(
      body,
      out_type=jax.ShapeDtypeStruct((n_chunks * _NP, 128), jnp.float32),
      mesh=mesh, scratch_types=scratch)


def _make_sc_counts():
  mesh = plsc.VectorSubcoreMesh(core_axis_name="c", subcore_axis_name="s")

  scratch = [
      pltpu.VMEM((_NBATCH, _K), jnp.int32),
      pltpu.VMEM((_K, 128), jnp.float32),
      pltpu.VMEM_SHARED((_NP, 128), jnp.float32),
      pltpu.SemaphoreType.DMA,
      pltpu.SemaphoreType.DMA,
  ]

  def body(dst2, zeros, ones, out2, dst_all, ones_v, acc, sem0, sem1):
    sid = lax.axis_index("s")
    r0 = sid * _ROWS
    sems = (sem0, sem1)

    pltpu.sync_copy(ones, ones_v)
    pltpu.sync_copy(dst2.at[pl.ds(sid * _NBATCH, _NBATCH)], dst_all)
    pltpu.sync_copy(zeros.at[pl.ds(r0, _ROWS)], acc.at[pl.ds(r0, _ROWS)])
    plsc.subcore_barrier()

    for j in range(2):
      pltpu.async_copy(ones_v, acc.at[dst_all.at[j]], sems[j], add=True)

    def step(g, carry):
      for j in range(2):
        b = 2 * g + j
        pltpu.make_async_copy(ones_v, acc.at[dst_all.at[0]], sems[j]).wait()
        pltpu.async_copy(ones_v, acc.at[dst_all.at[b + 2]], sems[j],
                         add=True)
      return carry

    lax.fori_loop(0, _NBATCH // 2 - 1, step, 0)
    for j in range(2):
      pltpu.make_async_copy(ones_v, acc.at[dst_all.at[0]], sems[j]).wait()
    plsc.subcore_barrier()
    pltpu.sync_copy(acc.at[pl.ds(r0, _ROWS)], out2.at[pl.ds(r0, _ROWS)])

  return pl.kernel(
      body,
      out_type=jax.ShapeDtypeStruct((_NP, 128), jnp.float32),
      mesh=mesh, scratch_types=scratch)


def _make_tc_layer(c_in, residual, n_blk=1024):
  d_in = c_in * 128
  grid = (_NP // n_blk,)

  def body(agg_r, x_r, cnt_r, wl_r, wr_r, b_r, out_r):
    inv = 1.0 / jnp.maximum(cnt_r[:, 0:1], 1.0)
    agg = jnp.concatenate([agg_r[i] for i in range(c_in)], axis=1)
    xin = jnp.concatenate([x_r[i] for i in range(c_in)], axis=1)
    z = jnp.dot(agg * inv, wl_r[...], preferred_element_type=jnp.float32)
    z = z + jnp.dot(xin, wr_r[...], preferred_element_type=jnp.float32)
    z = jnp.maximum(z + b_r[...], 0.0)
    if residual:
      z = z + xin
    for i in range(4):
      out_r[i] = z[:, 128 * i:128 * (i + 1)]

  in_specs = [
      pl.BlockSpec((c_in, n_blk, 128), lambda i: (0, i, 0)),
      pl.BlockSpec((c_in, n_blk, 128), lambda i: (0, i, 0)),
      pl.BlockSpec((n_blk, 128), lambda i: (i, 0)),
      pl.BlockSpec((d_in, 512), lambda i: (0, 0)),
      pl.BlockSpec((d_in, 512), lambda i: (0, 0)),
      pl.BlockSpec((1, 512), lambda i: (0, 0)),
  ]
  return pl.pallas_call(
      body,
      grid=grid,
      in_specs=in_specs,
      out_specs=pl.BlockSpec((4, n_blk, 128), lambda i: (0, i, 0)),
      out_shape=jax.ShapeDtypeStruct((4, _NP, 128), jnp.float32),
  )


def _make_tc_head():
  def body(h_r, w1_r, b1_r, w2_r, b2_r, w3_r, b3_r, out_r):
    s = jnp.sum(h_r[:, 0:_N, :], axis=1) * (1.0 / _N)
    z1 = jnp.zeros((1, 256), jnp.float32)
    for c in range(4):
      z1 = z1 + jnp.dot(s[c:c + 1], w1_r[c], preferred_element_type=jnp.float32)
    z1 = jnp.maximum(z1 + b1_r[...], 0.0)
    z2 = jnp.dot(z1, w2_r[...], preferred_element_type=jnp.float32)
    z2 = jnp.maximum(z2 + b2_r[...], 0.0)
    z3 = jnp.dot(z2, w3_r[...], preferred_element_type=jnp.float32) + b3_r[...]
    out_r[...] = 1.0 / (1.0 + jnp.exp(-z3))

  return pl.pallas_call(
      body, out_shape=jax.ShapeDtypeStruct((1, 1), jnp.float32))


_sc_agg2 = _make_sc_agg(2)
_sc_agg4 = _make_sc_agg(4)
_sc_counts = _make_sc_counts()
_tc_layer0 = _make_tc_layer(2, residual=False)
_tc_layer_res = _make_tc_layer(4, residual=True)
_tc_head = _make_tc_head()


def _chunk_offsets(src_p, n_chunks):
  flat = (src_p[None, :] +
          (jnp.arange(n_chunks, dtype=jnp.int32) * _NP)[:, None]).reshape(-1)
  return jnp.concatenate([flat, jnp.zeros((2 * _K,), jnp.int32)])


@jax.jit
def kernel(x, edge_index, Wl0, bl0, Wr0, Wl1, bl1, Wr1, Wl2, bl2, Wr2,
           Wc1, bc1, Wc2, bc2, Wc3, bc3):
  src = edge_index[0].astype(jnp.int32)
  dst = edge_index[1].astype(jnp.int32)
  pad = _EPAD - _E
  src_p = jnp.concatenate([src, jnp.zeros((pad,), jnp.int32)])
  dst_p = jnp.concatenate([dst, jnp.full((pad,), _N, jnp.int32)])
  src_off2 = _chunk_offsets(src_p, 2)
  src_off4 = _chunk_offsets(src_p, 4)

  x3 = x.reshape(_N, 2, 128).transpose(1, 0, 2)
  x3 = jnp.pad(x3, ((0, 0), (0, _NP - _N), (0, 0)))
  zeros = jnp.zeros((_NP, 128), jnp.float32)
  ones = jnp.ones((_K, 128), jnp.float32)

  dst2 = dst_p.reshape(_EPAD // _K, _K)
  cnt = _sc_counts(dst2, zeros, ones)
  agg0 = _sc_agg2(x3.reshape(2 * _NP, 128), src_off2, dst2, zeros)
  h = _tc_layer0(agg0.reshape(2, _NP, 128), x3, cnt, Wl0.T, Wr0.T,
                 bl0[None, :])

  agg1 = _sc_agg4(h.reshape(4 * _NP, 128), src_off4, dst2, zeros)
  h = _tc_layer_res(agg1.reshape(4, _NP, 128), h, cnt, Wl1.T, Wr1.T,
                    bl1[None, :])

  agg2 = _sc_agg4(h.reshape(4 * _NP, 128), src_off4, dst2, zeros)
  h = _tc_layer_res(agg2.reshape(4, _NP, 128), h, cnt, Wl2.T, Wr2.T,
                    bl2[None, :])

  return _tc_head(h, Wc1.T.reshape(4, 128, 256), bc1[None, :],
                  Wc2.T, bc2[None, :], Wc3.T, bc3[None, :])
